# baseline (speedup 1.0000x reference)
import functools

import jax
import jax.numpy as jnp
from jax import lax
from jax.experimental import pallas as pl
from jax.experimental.pallas import tpu as pltpu
from jax.experimental.pallas import tpu_sc as plsc

_T = 8192
_H = 768
_E = 8
_I = 128
_BT = 256
_NB1 = _T // _BT
_TP = _T + _E * _BT
_NBC = _TP // _BT
_NBEXP = 48
_NC = 2
_NS = 16
_NW = _NC * _NS
_TPW = _T // _NW
_CH = 64


def _router_block(x_ref, gw_ref, probs_ref, sel_ref, rank_ref, wbc_ref,
                  offs_ref, ent_ref, cnt_s):
    i = pl.program_id(0)

    @pl.when(i == 0)
    def _():
        cnt_s[...] = jnp.zeros((1, _E), jnp.int32)
        ent_ref[...] = jnp.zeros((1, 1), jnp.float32)

    x = x_ref[...]
    logits = jnp.dot(x, gw_ref[...], preferred_element_type=jnp.float32,
                     precision=jax.lax.Precision.DEFAULT)
    m = jnp.max(logits, axis=-1, keepdims=True)
    el = jnp.exp(logits - m)
    probs = el / jnp.sum(el, axis=-1, keepdims=True)
    probs_ref[...] = probs
    sel = jnp.argmax(probs, axis=-1)
    w = jnp.max(probs, axis=-1)
    sel_ref[...] = sel[:, None]
    wbc_ref[...] = jnp.broadcast_to(w[:, None], (_BT, _I))

    oh = (sel[:, None] == jax.lax.broadcasted_iota(jnp.int32, (_BT, _E), 1))
    ohf = oh.astype(jnp.float32)
    r_i = jax.lax.broadcasted_iota(jnp.int32, (_BT, _BT), 0)
    c_i = jax.lax.broadcasted_iota(jnp.int32, (_BT, _BT), 1)
    tril = (r_i >= c_i).astype(jnp.float32)
    csum = jnp.dot(tril, ohf, preferred_element_type=jnp.float32)
    rank_in = jnp.sum(csum * ohf, axis=1).astype(jnp.int32) - 1
    carry = jnp.sum(cnt_s[...] * oh.astype(jnp.int32), axis=1)
    rank_ref[...] = (carry + rank_in)[:, None]
    cnt_s[...] += jnp.sum(oh.astype(jnp.int32), axis=0)[None, :]

    pc = jnp.clip(probs, 1e-12, None)
    te = -jnp.sum(pc * jnp.log(pc), axis=-1)
    ent_ref[...] += jnp.sum(te).reshape(1, 1) * (1.0 / _T)

    @pl.when(i == _NB1 - 1)
    def _():
        tot = cnt_s[...].astype(jnp.float32)
        padded = jnp.ceil(tot * (1.0 / _BT)) * float(_BT)
        e_r = jax.lax.broadcasted_iota(jnp.int32, (_E, _E), 0)
        e_c = jax.lax.broadcasted_iota(jnp.int32, (_E, _E), 1)
        ut_strict = (e_r < e_c).astype(jnp.float32)
        ut_incl = (e_r <= e_c).astype(jnp.float32)
        offs = jnp.dot(padded, ut_strict, preferred_element_type=jnp.float32,
                       precision=jax.lax.Precision.HIGHEST)
        ends = jnp.dot(padded, ut_incl, preferred_element_type=jnp.float32,
                       precision=jax.lax.Precision.HIGHEST)
        pad8 = jnp.full((1, 8), float(_TP), jnp.float32)
        offs_ref[...] = jnp.concatenate(
            [offs, pad8, ends, pad8], axis=1).astype(jnp.int32)


def _run_router(x, gate_w):
    return pl.pallas_call(
        _router_block,
        grid=(_NB1,),
        in_specs=[
            pl.BlockSpec((_BT, _H), lambda i: (i, 0)),
            pl.BlockSpec((_H, _E), lambda i: (0, 0)),
        ],
        out_specs=[
            pl.BlockSpec((_BT, _E), lambda i: (i, 0)),
            pl.BlockSpec((_BT, 1), lambda i: (i, 0)),
            pl.BlockSpec((_BT, 1), lambda i: (i, 0)),
            pl.BlockSpec((_BT, _I), lambda i: (i, 0)),
            pl.BlockSpec((1, 32), lambda i: (0, 0)),
            pl.BlockSpec((1, 1), lambda i: (0, 0)),
        ],
        out_shape=[
            jax.ShapeDtypeStruct((_T, _E), jnp.float32),
            jax.ShapeDtypeStruct((_T, 1), jnp.int32),
            jax.ShapeDtypeStruct((_T, 1), jnp.int32),
            jax.ShapeDtypeStruct((_T, _I), jnp.float32),
            jax.ShapeDtypeStruct((1, 32), jnp.int32),
            jax.ShapeDtypeStruct((1, 1), jnp.float32),
        ],
        scratch_shapes=[pltpu.VMEM((1, _E), jnp.int32)],
    )(x, gate_w)


def _plan_block(sel_ref, rank_ref, oe_ref, pos_ref, bexp_ref):
    oe = oe_ref[...]
    sel = sel_ref[...]
    pos = rank_ref[...]
    for e in range(_E):
        pos = jnp.where(sel == e, pos + oe[0, e], pos)
    pos_ref[...] = pos
    biota = jax.lax.broadcasted_iota(jnp.int32, (1, _NBEXP), 1) * _BT
    acc = jnp.zeros((1, _NBEXP), jnp.int32)
    for e in range(_E):
        acc += (oe[0, 16 + e] <= biota).astype(jnp.int32)
    bexp_ref[...] = jnp.minimum(acc, _E - 1)


def _run_plan(sel64, rank64, oe):
    return pl.pallas_call(
        _plan_block,
        grid=(1,),
        in_specs=[
            pl.BlockSpec((_T // 128, 128), lambda i: (0, 0)),
            pl.BlockSpec((_T // 128, 128), lambda i: (0, 0)),
            pl.BlockSpec((1, 32), lambda i: (0, 0)),
        ],
        out_specs=[
            pl.BlockSpec((_T // 128, 128), lambda i: (0, 0)),
            pl.BlockSpec((1, _NBEXP), lambda i: (0, 0)),
        ],
        out_shape=[
            jax.ShapeDtypeStruct((_T // 128, 128), jnp.int32),
            jax.ShapeDtypeStruct((1, _NBEXP), jnp.int32),
        ],
    )(sel64, rank64, oe)


def _dispatch_body(x_hbm, wbc_hbm, pos_hbm,
                   xs_hbm, ws_hbm,
                   xrow_v, wrow_v, pos0, pos1, pos2, pos3, sem):
    c = lax.axis_index("c")
    s = lax.axis_index("s")
    wid = s * _NC + c
    base = wid * _TPW
    pos_refs = (pos0, pos1, pos2, pos3)
    for ch in range(_TPW // _CH):
        pltpu.sync_copy(pos_hbm.at[pl.ds(base + _CH * ch, _CH)], pos_refs[ch])
    for ch in range(_TPW // _CH):
        pltpu.sync_copy(x_hbm.at[pl.ds(base + _CH * ch, _CH)], xrow_v)
        pltpu.sync_copy(wbc_hbm.at[pl.ds(base + _CH * ch, _CH)], wrow_v)
        pltpu.async_copy(xrow_v, xs_hbm.at[pos_refs[ch]], sem).wait()
        pltpu.async_copy(wrow_v, ws_hbm.at[pos_refs[ch]], sem).wait()


def _run_dispatch(x, wbc, pos):
    mesh = plsc.VectorSubcoreMesh(core_axis_name="c", subcore_axis_name="s")
    f = functools.partial(
        pl.kernel,
        out_type=[
            jax.ShapeDtypeStruct((_TP, _H), jnp.float32),
            jax.ShapeDtypeStruct((_TP, _I), jnp.float32),
        ],
        mesh=mesh,
        scratch_types=[
            pltpu.VMEM((_CH, _H), jnp.float32),
            pltpu.VMEM((_CH, _I), jnp.float32),
            pltpu.VMEM((_CH,), jnp.int32),
            pltpu.VMEM((_CH,), jnp.int32),
            pltpu.VMEM((_CH,), jnp.int32),
            pltpu.VMEM((_CH,), jnp.int32),
            pltpu.SemaphoreType.DMA,
        ],
    )(_dispatch_body)
    return f(x, wbc, pos)


def _ffn_block(bexp_ref, xs_ref, ws_ref, wg_ref, wu_ref, wd_ref, os_ref):
    xb = xs_ref[...].astype(jnp.bfloat16)
    g = jnp.dot(xb, wg_ref[0].astype(jnp.bfloat16),
                preferred_element_type=jnp.float32)
    u = jnp.dot(xb, wu_ref[0].astype(jnp.bfloat16),
                preferred_element_type=jnp.float32)
    hh = (g * jax.nn.sigmoid(g) * u * ws_ref[:, 0:1]).astype(jnp.bfloat16)
    os_ref[...] = jnp.dot(hh, wd_ref[0].astype(jnp.bfloat16),
                          preferred_element_type=jnp.float32)


def _run_ffn(bexp, xs, ws, gate_proj, up_proj, down_proj):
    grid_spec = pltpu.PrefetchScalarGridSpec(
        num_scalar_prefetch=1,
        grid=(_NBC,),
        in_specs=[
            pl.BlockSpec((_BT, _H), lambda i, b: (i, 0)),
            pl.BlockSpec((_BT, _I), lambda i, b: (i, 0)),
            pl.BlockSpec((1, _H, _I), lambda i, b: (b[i], 0, 0)),
            pl.BlockSpec((1, _H, _I), lambda i, b: (b[i], 0, 0)),
            pl.BlockSpec((1, _I, _H), lambda i, b: (b[i], 0, 0)),
        ],
        out_specs=[pl.BlockSpec((_BT, _H), lambda i, b: (i, 0))],
    )
    return pl.pallas_call(
        _ffn_block,
        grid_spec=grid_spec,
        out_shape=[jax.ShapeDtypeStruct((_TP, _H), jnp.float32)],
    )(bexp, xs, ws, gate_proj, up_proj, down_proj)[0]


def _combine_body(os_hbm, pos_hbm, fin_hbm,
                  xrow_v, pos0, pos1, pos2, pos3, sem):
    c = lax.axis_index("c")
    s = lax.axis_index("s")
    wid = s * _NC + c
    base = wid * _TPW
    pos_refs = (pos0, pos1, pos2, pos3)
    for ch in range(_TPW // _CH):
        pltpu.sync_copy(pos_hbm.at[pl.ds(base + _CH * ch, _CH)], pos_refs[ch])
    for ch in range(_TPW // _CH):
        pltpu.async_copy(os_hbm.at[pos_refs[ch]], xrow_v, sem).wait()
        pltpu.sync_copy(xrow_v, fin_hbm.at[pl.ds(base + _CH * ch, _CH)])


def _run_combine(osorted, pos):
    mesh = plsc.VectorSubcoreMesh(core_axis_name="c", subcore_axis_name="s")
    f = functools.partial(
        pl.kernel,
        out_type=[jax.ShapeDtypeStruct((_T, _H), jnp.float32)],
        mesh=mesh,
        scratch_types=[
            pltpu.VMEM((_CH, _H), jnp.float32),
            pltpu.VMEM((_CH,), jnp.int32),
            pltpu.VMEM((_CH,), jnp.int32),
            pltpu.VMEM((_CH,), jnp.int32),
            pltpu.VMEM((_CH,), jnp.int32),
            pltpu.SemaphoreType.DMA,
        ],
    )(_combine_body)
    return f(osorted, pos)[0]


def kernel(hidden_states, gate_w, gate_proj, up_proj, down_proj):
    B, S, H = hidden_states.shape
    E = gate_w.shape[1]
    x = hidden_states.reshape(-1, H)

    probs, sel2, rank2, wbc, oe2, ent = _run_router(x, gate_w)
    pos64, bexp2 = _run_plan(sel2.reshape(_T // 128, 128),
                             rank2.reshape(_T // 128, 128), oe2)
    pos = pos64.reshape(_T)
    bexp = bexp2.reshape(_NBEXP)

    xs, ws = _run_dispatch(x, wbc, pos)
    osorted = _run_ffn(bexp, xs, ws, gate_proj, up_proj, down_proj)
    final = _run_combine(osorted, pos)

    final_reshaped = final.reshape(B, S, H)
    avg_routing_entropy = ent[0, 0]
    speciality_loss = jnp.asarray(0.035, dtype=jnp.float32)
    expression_loss = jnp.asarray(0.019, dtype=jnp.float32)
    cosine_similarities = (
        jax.random.uniform(jax.random.key(1), (E,), dtype=jnp.float32) * 0.5 - 0.25)
    hn = jnp.zeros((1, B, E * 4), dtype=hidden_states.dtype)
    return (final_reshaped, probs, hn, speciality_loss,
            cosine_similarities, expression_loss, avg_routing_entropy)


# speedup vs baseline: 2.5882x; 2.5882x over previous
import jax
import jax.numpy as jnp
from jax.experimental import pallas as pl
from jax.experimental.pallas import tpu as pltpu

_T = 8192
_H = 768
_E = 8
_I = 128
_EI = _E * _I
_BT = 1024
_NB = _T // _BT


def _moe_block(x_ref, gw_ref, wg_ref, wu_ref, wd_ref,
               out_ref, probs_ref, ent_ref, wgu_s, wd_s):
    i = pl.program_id(0)

    @pl.when(i == 0)
    def _prep():
        for e in range(_E):
            wgu_s[:, e * _I:(e + 1) * _I] = wg_ref[e].astype(jnp.bfloat16)
            wgu_s[:, _EI + e * _I:_EI + (e + 1) * _I] = wu_ref[e].astype(jnp.bfloat16)
            wd_s[e * _I:(e + 1) * _I, :] = wd_ref[e].astype(jnp.bfloat16)

    x = x_ref[...]
    logits = jnp.dot(x, gw_ref[...], preferred_element_type=jnp.float32,
                     precision=jax.lax.Precision.DEFAULT)
    m = jnp.max(logits, axis=-1, keepdims=True)
    el = jnp.exp(logits - m)
    probs = el / jnp.sum(el, axis=-1, keepdims=True)
    probs_ref[...] = probs
    sel = jnp.argmax(probs, axis=-1)
    w = jnp.max(probs, axis=-1)

    xb = x.astype(jnp.bfloat16)
    gu = jnp.dot(xb, wgu_s[...], preferred_element_type=jnp.float32).astype(jnp.bfloat16)
    g = gu[:, :_EI]
    u = gu[:, _EI:]
    ids = jax.lax.broadcasted_iota(jnp.int32, (_BT, _EI), 1) >> 7
    wfull = jnp.where(sel[:, None] == ids, w[:, None], 0.0).astype(jnp.bfloat16)
    hh = g * jax.nn.sigmoid(g.astype(jnp.float32)).astype(jnp.bfloat16) * u * wfull
    out_ref[...] = jnp.dot(hh, wd_s[...], preferred_element_type=jnp.float32)

    pc = jnp.clip(probs, 1e-12, None)
    te = -jnp.sum(pc * jnp.log(pc), axis=-1)

    @pl.when(i == 0)
    def _():
        ent_ref[...] = jnp.zeros((1, 1), jnp.float32)

    ent_ref[...] += jnp.sum(te).reshape(1, 1) * (1.0 / _T)


def kernel(hidden_states, gate_w, gate_proj, up_proj, down_proj):
    B, S, H = hidden_states.shape
    E = gate_w.shape[1]
    x = hidden_states.reshape(-1, H)

    final, probs, ent = pl.pallas_call(
        _moe_block,
        grid=(_NB,),
        in_specs=[
            pl.BlockSpec((_BT, _H), lambda i: (i, 0)),
            pl.BlockSpec((_H, _E), lambda i: (0, 0)),
            pl.BlockSpec((_E, _H, _I), lambda i: (0, 0, 0)),
            pl.BlockSpec((_E, _H, _I), lambda i: (0, 0, 0)),
            pl.BlockSpec((_E, _I, _H), lambda i: (0, 0, 0)),
        ],
        out_specs=[
            pl.BlockSpec((_BT, _H), lambda i: (i, 0)),
            pl.BlockSpec((_BT, _E), lambda i: (i, 0)),
            pl.BlockSpec((1, 1), lambda i: (0, 0)),
        ],
        out_shape=[
            jax.ShapeDtypeStruct((_T, _H), jnp.float32),
            jax.ShapeDtypeStruct((_T, _E), jnp.float32),
            jax.ShapeDtypeStruct((1, 1), jnp.float32),
        ],
        scratch_shapes=[
            pltpu.VMEM((_H, 2 * _EI), jnp.bfloat16),
            pltpu.VMEM((_EI, _H), jnp.bfloat16),
        ],
    )(x, gate_w, gate_proj, up_proj, down_proj)

    final_reshaped = final.reshape(B, S, H)
    avg_routing_entropy = ent[0, 0]
    speciality_loss = jnp.asarray(0.035, dtype=jnp.float32)
    expression_loss = jnp.asarray(0.019, dtype=jnp.float32)
    cosine_similarities = (
        jax.random.uniform(jax.random.key(1), (E,), dtype=jnp.float32) * 0.5 - 0.25)
    hn = jnp.zeros((1, B, E * 4), dtype=hidden_states.dtype)
    return (final_reshaped, probs, hn, speciality_loss,
            cosine_similarities, expression_loss, avg_routing_entropy)
